# baseline (speedup 1.0000x reference)
import functools

import jax
import jax.numpy as jnp
from jax import lax
from jax.experimental import pallas as pl
from jax.experimental.pallas import tpu as pltpu
from jax.experimental.pallas import tpu_sc as plsc

_NUM_CLASSES = 1000000
_EMBED_DIM = 64
_BATCH = 16384
_DROPOUT_P = 0.1

_info = plsc.get_sparse_core_info()
_NC, _NS = _info.num_cores, _info.num_subcores
_NW = _NC * _NS
_BPW = _BATCH // _NW
_CHUNK = 128
_NCHUNK = _BPW // _CHUNK

_mesh = plsc.VectorSubcoreMesh(core_axis_name="c", subcore_axis_name="s")


@functools.partial(
    pl.kernel,
    mesh=_mesh,
    out_type=jax.ShapeDtypeStruct((_BATCH, _EMBED_DIM), jnp.float32),
    compiler_params=pltpu.CompilerParams(use_tc_tiling_on_sc=False),
    scratch_types=[
        pltpu.VMEM((_NCHUNK, _CHUNK), jnp.int32),
        pltpu.VMEM((_BPW, _EMBED_DIM), jnp.float32),
        pltpu.SemaphoreType.DMA,
    ],
)
def _embed_gather(table_hbm, idx_hbm, out_hbm, idx_v, rows_v, sem):
    wid = lax.axis_index("s") * _NC + lax.axis_index("c")
    base = wid * _BPW
    pltpu.sync_copy(idx_hbm.at[wid], idx_v)
    copies = [
        pltpu.async_copy(
            table_hbm.at[idx_v.at[j]],
            rows_v.at[pl.ds(j * _CHUNK, _CHUNK)],
            sem,
        )
        for j in range(_NCHUNK)
    ]
    for c in copies:
        c.wait()
    pltpu.sync_copy(rows_v, out_hbm.at[pl.ds(base, _BPW)])


def kernel(labels, codebook, training):
    drop_ids = jax.random.normal(jax.random.key(42), (labels.shape[0],)) < _DROPOUT_P
    dropped = jnp.where(drop_ids, _NUM_CLASSES, labels)
    eff = jnp.where(jnp.asarray(training) != 0, dropped, labels)
    idx = eff.reshape(_NW, _NCHUNK, _CHUNK)
    return _embed_gather(codebook, idx)


# speedup vs baseline: 1.7191x; 1.7191x over previous
import functools

import jax
import jax.numpy as jnp
from jax import lax
from jax.experimental import pallas as pl
from jax.experimental.pallas import tpu as pltpu
from jax.experimental.pallas import tpu_sc as plsc

_NUM_CLASSES = 1000000
_EMBED_DIM = 64
_BATCH = 16384
_DROPOUT_P = 0.1

_info = plsc.get_sparse_core_info()
_NC, _NS = _info.num_cores, _info.num_subcores
_NW = _NC * _NS
_BPW = _BATCH // _NW

_mesh = plsc.VectorSubcoreMesh(core_axis_name="c", subcore_axis_name="s")


@functools.partial(
    pl.kernel,
    mesh=_mesh,
    out_type=jax.ShapeDtypeStruct((_BATCH, _EMBED_DIM), jnp.float32),
    scratch_types=[
        pltpu.VMEM((_BPW,), jnp.int32),
        pltpu.VMEM((_BPW, _EMBED_DIM), jnp.float32),
        pltpu.SemaphoreType.DMA,
        pltpu.SemaphoreType.DMA,
    ],
)
def _embed_gather(table_hbm, idx_hbm, out_hbm, idx_v, rows_v, sem_i, sem_g):
    wid = lax.axis_index("s") * _NC + lax.axis_index("c")
    base = wid * _BPW
    pltpu.async_copy(idx_hbm.at[pl.ds(base, _BPW)], idx_v, sem_i).wait()

    def body(g, _):
        vec = idx_v[pl.ds(g * 16, 16)]
        for j in range(16):
            pltpu.async_copy(
                table_hbm.at[pl.ds(vec[j], 1)],
                rows_v.at[pl.ds(g * 16 + j, 1)],
                sem_g,
            )
        return ()

    lax.fori_loop(0, _BPW // 16, body, ())
    pltpu.make_async_copy(table_hbm.at[pl.ds(0, _BPW)], rows_v, sem_g).wait()
    pltpu.sync_copy(rows_v, out_hbm.at[pl.ds(base, _BPW)])


def kernel(labels, codebook, training):
    drop_ids = jax.random.normal(jax.random.key(42), (labels.shape[0],)) < _DROPOUT_P
    dropped = jnp.where(drop_ids, _NUM_CLASSES, labels)
    eff = jnp.where(jnp.asarray(training) != 0, dropped, labels)
    return _embed_gather(codebook, eff)


# speedup vs baseline: 2.3424x; 1.3626x over previous
import functools

import jax
import jax.numpy as jnp
from jax import lax
from jax.experimental import pallas as pl
from jax.experimental.pallas import tpu as pltpu
from jax.experimental.pallas import tpu_sc as plsc

_NUM_CLASSES = 1000000
_EMBED_DIM = 64
_BATCH = 16384
_DROPOUT_P = 0.1

_info = plsc.get_sparse_core_info()
_NC, _NS = _info.num_cores, _info.num_subcores
_NW = _NC * _NS
_BPW = _BATCH // _NW
_GRP = 8
_NGRP = _BPW // _GRP

_mesh = plsc.VectorSubcoreMesh(core_axis_name="c", subcore_axis_name="s")


@functools.partial(
    pl.kernel,
    mesh=_mesh,
    compiler_params=pltpu.CompilerParams(
        disable_bounds_checks=True, needs_layout_passes=False
    ),
    out_type=jax.ShapeDtypeStruct((_EMBED_DIM, _BATCH), jnp.float32),
    scratch_types=[
        pltpu.VMEM((_BPW + 16,), jnp.int32),
        pltpu.VMEM((_GRP * _EMBED_DIM, 128), jnp.float32),
        pltpu.VMEM((_EMBED_DIM, _BPW), jnp.float32),
        pltpu.SemaphoreType.DMA,
        pltpu.SemaphoreType.DMA,
    ],
)
def _embed_gather(table_hbm, idx_hbm, out_hbm, idx_v, blocks_v, stage_v,
                  sem_i, sem_g):
    wid = lax.axis_index("s") * _NC + lax.axis_index("c")
    base = wid * _BPW
    pltpu.async_copy(idx_hbm.at[pl.ds(base, _BPW)],
                     idx_v.at[pl.ds(0, _BPW)], sem_i).wait()

    def body(g, _):
        vec = idx_v[pl.ds(g * _GRP, 16)]
        copies = []
        for b in range(_GRP):
            lbl = vec[b]
            col0 = pl.multiple_of((lbl >> 7) * 128, 128)
            copies.append(pltpu.async_copy(
                table_hbm.at[:, pl.ds(col0, 128)],
                blocks_v.at[pl.ds(b * _EMBED_DIM, _EMBED_DIM), :],
                sem_g,
            ))
        for c in copies:
            c.wait()
        for b in range(_GRP):
            lane = jnp.full((16,), vec[b] & 127, dtype=jnp.int32)
            pos = jnp.full((16,), g * _GRP + b, dtype=jnp.int32)
            for k in range(_EMBED_DIM // 16):
                rows = lax.iota(jnp.int32, 16) + (b * _EMBED_DIM + k * 16)
                val = plsc.load_gather(blocks_v, [rows, lane])
                out_rows = lax.iota(jnp.int32, 16) + k * 16
                plsc.store_scatter(stage_v, [out_rows, pos], val)
        return ()

    lax.fori_loop(0, _NGRP, body, ())
    pltpu.sync_copy(stage_v, out_hbm.at[:, pl.ds(base, _BPW)])


def kernel(labels, codebook, training):
    drop_ids = jax.random.normal(jax.random.key(42), (labels.shape[0],)) < _DROPOUT_P
    dropped = jnp.where(drop_ids, _NUM_CLASSES, labels)
    eff = jnp.where(jnp.asarray(training) != 0, dropped, labels)
    table_t = lax.optimization_barrier(codebook.T)
    out_t = _embed_gather(table_t, eff)
    return out_t.T


# speedup vs baseline: 2.5057x; 1.0697x over previous
import functools

import jax
import jax.numpy as jnp
from jax import lax
from jax.experimental import pallas as pl
from jax.experimental.pallas import tpu as pltpu
from jax.experimental.pallas import tpu_sc as plsc

_NUM_CLASSES = 1000000
_EMBED_DIM = 64
_BATCH = 16384
_DROPOUT_P = 0.1

_info = plsc.get_sparse_core_info()
_NC, _NS = _info.num_cores, _info.num_subcores
_NW = _NC * _NS
_BPW = _BATCH // _NW
_GRP = 4
_NPAIR = _BPW // (2 * _GRP)

_mesh = plsc.VectorSubcoreMesh(core_axis_name="c", subcore_axis_name="s")


@functools.partial(
    pl.kernel,
    mesh=_mesh,
    compiler_params=pltpu.CompilerParams(
        disable_bounds_checks=True, needs_layout_passes=False
    ),
    out_type=jax.ShapeDtypeStruct((_EMBED_DIM, _BATCH), jnp.float32),
    scratch_types=[
        pltpu.VMEM((_BPW + 16,), jnp.int32),
        pltpu.VMEM((_GRP * _EMBED_DIM, 128), jnp.float32),
        pltpu.VMEM((_GRP * _EMBED_DIM, 128), jnp.float32),
        pltpu.VMEM((_EMBED_DIM, _BPW), jnp.float32),
        pltpu.SemaphoreType.DMA,
        pltpu.SemaphoreType.DMA,
        pltpu.SemaphoreType.DMA,
    ],
)
def _embed_gather(table_hbm, idx_hbm, out_hbm, idx_v, buf_a, buf_b, stage_v,
                  sem_i, sem_a, sem_b):
    wid = lax.axis_index("s") * _NC + lax.axis_index("c")
    base = wid * _BPW
    pltpu.async_copy(idx_hbm.at[pl.ds(base, _BPW)],
                     idx_v.at[pl.ds(0, _BPW)], sem_i).wait()
    idx_v[pl.ds(_BPW, 16)] = jnp.zeros((16,), jnp.int32)

    def fire(first_label, buf, sem):
        vec = idx_v[pl.ds(first_label, 16)]
        for b in range(_GRP):
            col0 = pl.multiple_of((vec[b] >> 7) * 128, 128)
            pltpu.async_copy(
                table_hbm.at[:, pl.ds(col0, 128)],
                buf.at[pl.ds(b * _EMBED_DIM, _EMBED_DIM), :],
                sem,
            )

    def drain(buf, sem):
        for b in range(_GRP):
            pltpu.make_async_copy(
                table_hbm.at[:, pl.ds(0, 128)],
                buf.at[pl.ds(b * _EMBED_DIM, _EMBED_DIM), :],
                sem,
            ).wait()

    def extract(first_label, buf):
        vec = idx_v[pl.ds(first_label, 16)]
        for b in range(_GRP):
            lane = jnp.full((16,), vec[b] & 127, dtype=jnp.int32)
            pos = jnp.full((16,), first_label + b, dtype=jnp.int32)
            for k in range(_EMBED_DIM // 16):
                rows = lax.iota(jnp.int32, 16) + (b * _EMBED_DIM + k * 16)
                val = plsc.load_gather(buf, [rows, lane])
                out_rows = lax.iota(jnp.int32, 16) + k * 16
                plsc.store_scatter(stage_v, [out_rows, pos], val)

    fire(0, buf_a, sem_a)
    fire(_GRP, buf_b, sem_b)

    def body(i, _):
        la = i * 2 * _GRP
        nxt_a = jnp.minimum(la + 2 * _GRP, _BPW - 2 * _GRP)
        nxt_b = nxt_a + _GRP
        drain(buf_a, sem_a)
        extract(la, buf_a)
        fire(nxt_a, buf_a, sem_a)
        drain(buf_b, sem_b)
        extract(la + _GRP, buf_b)
        fire(nxt_b, buf_b, sem_b)
        return ()

    lax.fori_loop(0, _NPAIR - 1, body, ())
    la = (_NPAIR - 1) * 2 * _GRP
    drain(buf_a, sem_a)
    extract(la, buf_a)
    drain(buf_b, sem_b)
    extract(la + _GRP, buf_b)
    pltpu.sync_copy(stage_v, out_hbm.at[:, pl.ds(base, _BPW)])


def kernel(labels, codebook, training):
    drop_ids = jax.random.normal(jax.random.key(42), (labels.shape[0],)) < _DROPOUT_P
    dropped = jnp.where(drop_ids, _NUM_CLASSES, labels)
    eff = jnp.where(jnp.asarray(training) != 0, dropped, labels)
    table_t = lax.optimization_barrier(codebook.T)
    out_t = _embed_gather(table_t, eff)
    return out_t.T


# speedup vs baseline: 2.7995x; 1.1173x over previous
import functools

import jax
import jax.numpy as jnp
from jax import lax
from jax.experimental import pallas as pl
from jax.experimental.pallas import tpu as pltpu
from jax.experimental.pallas import tpu_sc as plsc

_NUM_CLASSES = 1000000
_EMBED_DIM = 64
_BATCH = 16384
_DROPOUT_P = 0.1

_info = plsc.get_sparse_core_info()
_NC, _NS = _info.num_cores, _info.num_subcores
_NW = _NC * _NS
_BPW = _BATCH // _NW
_GRP = 2
_NBUF = 4
_NGRP = _BPW // _GRP
_NIT = _NGRP // _NBUF - 1

_mesh = plsc.VectorSubcoreMesh(core_axis_name="c", subcore_axis_name="s")


@functools.partial(
    pl.kernel,
    mesh=_mesh,
    compiler_params=pltpu.CompilerParams(
        disable_bounds_checks=True, needs_layout_passes=False
    ),
    out_type=jax.ShapeDtypeStruct((_EMBED_DIM, _BATCH), jnp.float32),
    scratch_types=[
        pltpu.VMEM((_BPW + 16,), jnp.int32),
        pltpu.VMEM((_GRP * _EMBED_DIM, 128), jnp.float32),
        pltpu.VMEM((_GRP * _EMBED_DIM, 128), jnp.float32),
        pltpu.VMEM((_GRP * _EMBED_DIM, 128), jnp.float32),
        pltpu.VMEM((_GRP * _EMBED_DIM, 128), jnp.float32),
        pltpu.VMEM((_EMBED_DIM, _BPW), jnp.float32),
        pltpu.SemaphoreType.DMA,
        pltpu.SemaphoreType.DMA,
        pltpu.SemaphoreType.DMA,
        pltpu.SemaphoreType.DMA,
        pltpu.SemaphoreType.DMA,
    ],
)
def _embed_gather(table_hbm, idx_hbm, out_hbm, idx_v, buf_a, buf_b, buf_c,
                  buf_d, stage_v, sem_i, sem_a, sem_b, sem_c, sem_d):
    wid = lax.axis_index("s") * _NC + lax.axis_index("c")
    base = wid * _BPW
    pltpu.async_copy(idx_hbm.at[pl.ds(base, _BPW)],
                     idx_v.at[pl.ds(0, _BPW)], sem_i).wait()
    idx_v[pl.ds(_BPW, 16)] = jnp.zeros((16,), jnp.int32)

    def fire(first_label, buf, sem):
        vec = idx_v[pl.ds(first_label, 16)]
        for b in range(_GRP):
            col0 = pl.multiple_of((vec[b] >> 7) * 128, 128)
            pltpu.async_copy(
                table_hbm.at[:, pl.ds(col0, 128)],
                buf.at[pl.ds(b * _EMBED_DIM, _EMBED_DIM), :],
                sem,
            )

    def drain(buf, sem):
        for b in range(_GRP):
            pltpu.make_async_copy(
                table_hbm.at[:, pl.ds(0, 128)],
                buf.at[pl.ds(b * _EMBED_DIM, _EMBED_DIM), :],
                sem,
            ).wait()

    def extract(first_label, buf):
        vec = idx_v[pl.ds(first_label, 16)]
        for b in range(_GRP):
            lane = jnp.full((16,), vec[b] & 127, dtype=jnp.int32)
            pos = jnp.full((16,), first_label + b, dtype=jnp.int32)
            for k in range(_EMBED_DIM // 16):
                rows = lax.iota(jnp.int32, 16) + (b * _EMBED_DIM + k * 16)
                val = plsc.load_gather(buf, [rows, lane])
                out_rows = lax.iota(jnp.int32, 16) + k * 16
                plsc.store_scatter(stage_v, [out_rows, pos], val)

    ring = ((buf_a, sem_a), (buf_b, sem_b), (buf_c, sem_c), (buf_d, sem_d))
    for off, (buf, sem) in enumerate(ring):
        fire(off * _GRP, buf, sem)

    def body(i, _):
        g0 = i * _NBUF
        for off, (buf, sem) in enumerate(ring):
            la = (g0 + off) * _GRP
            drain(buf, sem)
            extract(la, buf)
            fire(la + _NBUF * _GRP, buf, sem)
        return ()

    lax.fori_loop(0, _NIT, body, ())
    g0 = _NIT * _NBUF
    for off, (buf, sem) in enumerate(ring):
        drain(buf, sem)
        extract((g0 + off) * _GRP, buf)
    pltpu.sync_copy(stage_v, out_hbm.at[:, pl.ds(base, _BPW)])


def kernel(labels, codebook, training):
    drop_ids = jax.random.normal(jax.random.key(42), (labels.shape[0],)) < _DROPOUT_P
    dropped = jnp.where(drop_ids, _NUM_CLASSES, labels)
    eff = jnp.where(jnp.asarray(training) != 0, dropped, labels)
    table_t = lax.optimization_barrier(codebook.T)
    out_t = _embed_gather(table_t, eff)
    return out_t.T


# speedup vs baseline: 3.0231x; 1.0799x over previous
import functools

import jax
import jax.numpy as jnp
from jax import lax
from jax.experimental import pallas as pl
from jax.experimental.pallas import tpu as pltpu
from jax.experimental.pallas import tpu_sc as plsc

_NUM_CLASSES = 1000000
_EMBED_DIM = 64
_BATCH = 16384
_DROPOUT_P = 0.1

_info = plsc.get_sparse_core_info()
_NC, _NS = _info.num_cores, _info.num_subcores
_NW = _NC * _NS
_BPW = _BATCH // _NW
_GRP = 1
_NBUF = 8
_NGRP = _BPW // _GRP
_NIT = _NGRP // _NBUF - 1

_mesh = plsc.VectorSubcoreMesh(core_axis_name="c", subcore_axis_name="s")


@functools.partial(
    pl.kernel,
    mesh=_mesh,
    compiler_params=pltpu.CompilerParams(
        disable_bounds_checks=True, needs_layout_passes=False
    ),
    out_type=jax.ShapeDtypeStruct((_EMBED_DIM, _BATCH), jnp.float32),
    scratch_types=(
        [pltpu.VMEM((_BPW + 16,), jnp.int32)]
        + [pltpu.VMEM((_GRP * _EMBED_DIM, 128), jnp.float32)] * _NBUF
        + [pltpu.VMEM((_EMBED_DIM, _BPW), jnp.float32)]
        + [pltpu.SemaphoreType.DMA] * (_NBUF + 1)
    ),
)
def _embed_gather(table_hbm, idx_hbm, out_hbm, idx_v, *rest):
    bufs = rest[:_NBUF]
    stage_v = rest[_NBUF]
    sem_i = rest[_NBUF + 1]
    sems = rest[_NBUF + 2:]
    wid = lax.axis_index("s") * _NC + lax.axis_index("c")
    base = wid * _BPW
    pltpu.async_copy(idx_hbm.at[pl.ds(base, _BPW)],
                     idx_v.at[pl.ds(0, _BPW)], sem_i).wait()
    idx_v[pl.ds(_BPW, 16)] = jnp.zeros((16,), jnp.int32)

    def fire(first_label, buf, sem):
        vec = idx_v[pl.ds(first_label, 16)]
        for b in range(_GRP):
            col0 = pl.multiple_of((vec[b] >> 7) * 128, 128)
            pltpu.async_copy(
                table_hbm.at[:, pl.ds(col0, 128)],
                buf.at[pl.ds(b * _EMBED_DIM, _EMBED_DIM), :],
                sem,
            )

    def drain(buf, sem):
        for b in range(_GRP):
            pltpu.make_async_copy(
                table_hbm.at[:, pl.ds(0, 128)],
                buf.at[pl.ds(b * _EMBED_DIM, _EMBED_DIM), :],
                sem,
            ).wait()

    def extract(first_label, buf):
        vec = idx_v[pl.ds(first_label, 16)]
        for b in range(_GRP):
            lane = jnp.full((16,), vec[b] & 127, dtype=jnp.int32)
            pos = jnp.full((16,), first_label + b, dtype=jnp.int32)
            for k in range(_EMBED_DIM // 16):
                rows = lax.iota(jnp.int32, 16) + (b * _EMBED_DIM + k * 16)
                val = plsc.load_gather(buf, [rows, lane])
                out_rows = lax.iota(jnp.int32, 16) + k * 16
                plsc.store_scatter(stage_v, [out_rows, pos], val)

    ring = tuple(zip(bufs, sems))
    for off, (buf, sem) in enumerate(ring):
        fire(off * _GRP, buf, sem)

    def body(i, _):
        g0 = i * _NBUF
        for off, (buf, sem) in enumerate(ring):
            la = (g0 + off) * _GRP
            drain(buf, sem)
            extract(la, buf)
            fire(la + _NBUF * _GRP, buf, sem)
        return ()

    lax.fori_loop(0, _NIT, body, ())
    g0 = _NIT * _NBUF
    for off, (buf, sem) in enumerate(ring):
        drain(buf, sem)
        extract((g0 + off) * _GRP, buf)
    pltpu.sync_copy(stage_v, out_hbm.at[:, pl.ds(base, _BPW)])


def kernel(labels, codebook, training):
    drop_ids = jax.random.normal(jax.random.key(42), (labels.shape[0],)) < _DROPOUT_P
    dropped = jnp.where(drop_ids, _NUM_CLASSES, labels)
    eff = jnp.where(jnp.asarray(training) != 0, dropped, labels)
    table_t = lax.optimization_barrier(codebook.T)
    out_t = _embed_gather(table_t, eff)
    return out_t.T


# speedup vs baseline: 3.0263x; 1.0010x over previous
import functools

import jax
import jax.numpy as jnp
from jax import lax
from jax.experimental import pallas as pl
from jax.experimental.pallas import tpu as pltpu
from jax.experimental.pallas import tpu_sc as plsc

_NUM_CLASSES = 1000000
_EMBED_DIM = 64
_BATCH = 16384
_DROPOUT_P = 0.1

_info = plsc.get_sparse_core_info()
_NC, _NS = _info.num_cores, _info.num_subcores
_NW = _NC * _NS
_BPW = _BATCH // _NW
_GRP = 1
_NBUF = 10
_G = 520
_NIT = _G // _NBUF - 1

_mesh = plsc.VectorSubcoreMesh(core_axis_name="c", subcore_axis_name="s")


@functools.partial(
    pl.kernel,
    mesh=_mesh,
    compiler_params=pltpu.CompilerParams(
        disable_bounds_checks=True, needs_layout_passes=False
    ),
    out_type=jax.ShapeDtypeStruct((_EMBED_DIM, _BATCH), jnp.float32),
    scratch_types=(
        [pltpu.VMEM((_BPW + 16,), jnp.int32)]
        + [pltpu.VMEM((_GRP * _EMBED_DIM, 128), jnp.float32)] * _NBUF
        + [pltpu.VMEM((_EMBED_DIM, _BPW), jnp.float32)]
        + [pltpu.SemaphoreType.DMA] * (_NBUF + 1)
    ),
)
def _embed_gather(table_hbm, idx_hbm, out_hbm, idx_v, *rest):
    bufs = rest[:_NBUF]
    stage_v = rest[_NBUF]
    sem_i = rest[_NBUF + 1]
    sems = rest[_NBUF + 2:]
    wid = lax.axis_index("s") * _NC + lax.axis_index("c")
    base = wid * _BPW
    pltpu.async_copy(idx_hbm.at[pl.ds(base, _BPW)],
                     idx_v.at[pl.ds(0, _BPW)], sem_i).wait()
    idx_v[pl.ds(_BPW, 16)] = jnp.zeros((16,), jnp.int32)

    def fire(first_label, buf, sem):
        vec = idx_v[pl.ds(first_label, 16)]
        for b in range(_GRP):
            col0 = pl.multiple_of((vec[b] >> 7) * 128, 128)
            pltpu.async_copy(
                table_hbm.at[:, pl.ds(col0, 128)],
                buf.at[pl.ds(b * _EMBED_DIM, _EMBED_DIM), :],
                sem,
            )

    def drain(buf, sem):
        for b in range(_GRP):
            pltpu.make_async_copy(
                table_hbm.at[:, pl.ds(0, 128)],
                buf.at[pl.ds(b * _EMBED_DIM, _EMBED_DIM), :],
                sem,
            ).wait()

    def extract(first_label, buf):
        vec = idx_v[pl.ds(first_label, 16)]
        for b in range(_GRP):
            lane = jnp.full((16,), vec[b] & 127, dtype=jnp.int32)
            pos = jnp.full((16,), first_label + b, dtype=jnp.int32)
            for k in range(_EMBED_DIM // 16):
                rows = lax.iota(jnp.int32, 16) + (b * _EMBED_DIM + k * 16)
                val = plsc.load_gather(buf, [rows, lane])
                out_rows = lax.iota(jnp.int32, 16) + k * 16
                plsc.store_scatter(stage_v, [out_rows, pos], val)

    ring = tuple(zip(bufs, sems))
    for off, (buf, sem) in enumerate(ring):
        fire(off * _GRP, buf, sem)

    def body(i, _):
        g0 = i * _NBUF
        for off, (buf, sem) in enumerate(ring):
            la = (g0 + off) * _GRP
            drain(buf, sem)
            extract(jnp.minimum(la, _BPW - _GRP), buf)
            fire(jnp.minimum(la + _NBUF * _GRP, _BPW - _GRP), buf, sem)
        return ()

    lax.fori_loop(0, _NIT, body, ())
    g0 = _NIT * _NBUF
    for off, (buf, sem) in enumerate(ring):
        drain(buf, sem)
        extract(min((g0 + off) * _GRP, _BPW - _GRP), buf)
    pltpu.sync_copy(stage_v, out_hbm.at[:, pl.ds(base, _BPW)])


def kernel(labels, codebook, training):
    drop_ids = jax.random.normal(jax.random.key(42), (labels.shape[0],)) < _DROPOUT_P
    dropped = jnp.where(drop_ids, _NUM_CLASSES, labels)
    eff = jnp.where(jnp.asarray(training) != 0, dropped, labels)
    table_t = lax.optimization_barrier(codebook.T)
    out_t = _embed_gather(table_t, eff)
    return out_t.T
